# Initial kernel scaffold; baseline (speedup 1.0000x reference)
import jax
import jax.numpy as jnp
from jax.experimental import pallas as pl


def kernel(x, edge_index, batch, W1, b1, W2, b2, Wg1, bg1, Wg2, bg2):
    raise NotImplementedError("write your pallas kernel here")


# speedup vs baseline: 4.4753x; 4.4753x over previous
import functools

import jax
import jax.numpy as jnp
from jax import lax
from jax.experimental import pallas as pl
from jax.experimental.pallas import tpu as pltpu
from jax.experimental.pallas import tpu_sc as plsc

N = 50000
E = 800000
G = 128
D = 84
D2 = 840
DP = 96
C = 16
NCHUNK = DP // C
N_PAD = 50048
BLK = 128
SUPER = 24
E_PAD = 835584
ROWS_PER_TILE = 408
NSUPER = 17
ZROWS = N_PAD // 16


def _sc_agg_body(t0, t1, t2, t3, t4, t5, srcv, dstv,
                 o0, o1, o2, o3, o4, o5,
                 acc, src_buf, dst_buf, rb0, rb1, zbuf, sem0, sem1):
    c = lax.axis_index("c")
    s = lax.axis_index("s")

    def chunk_pass(table_ref, out_ref):
        def zero_row(i, _):
            zbuf[i] = jnp.zeros((C,), jnp.float32)
            return 0
        lax.fori_loop(0, ZROWS, zero_row, 0)
        pltpu.sync_copy(zbuf, acc.at[pl.ds(s * ZROWS, ZROWS)])
        plsc.subcore_barrier()

        rbufs = (rb0, rb1)
        sems = (sem0, sem1)

        def super_body(sb, _):
            row0 = s * ROWS_PER_TILE + sb * SUPER
            pltpu.sync_copy(srcv.at[pl.ds(row0, SUPER)], src_buf)
            pltpu.sync_copy(dstv.at[pl.ds(row0, SUPER)], dst_buf)
            descs = {0: pltpu.async_copy(table_ref.at[src_buf.at[0]],
                                         rb0, sem0)}
            for j in range(SUPER):
                if j + 1 < SUPER:
                    descs[j + 1] = pltpu.async_copy(
                        table_ref.at[src_buf.at[j + 1]],
                        rbufs[(j + 1) % 2], sems[(j + 1) % 2])
                descs[j].wait()
                pltpu.sync_copy(rbufs[j % 2], acc.at[dst_buf.at[j]], add=True)
            return 0

        lax.fori_loop(0, NSUPER, super_body, 0)
        plsc.subcore_barrier()
        pltpu.sync_copy(acc.at[pl.ds(s * ZROWS, ZROWS)], zbuf)
        pltpu.sync_copy(zbuf, out_ref.at[pl.ds(s * ZROWS, ZROWS)])
        plsc.subcore_barrier()

    @pl.when(c == 0)
    def _():
        chunk_pass(t0, o0)
        chunk_pass(t1, o1)
        chunk_pass(t2, o2)

    @pl.when(c == 1)
    def _():
        chunk_pass(t3, o3)
        chunk_pass(t4, o4)
        chunk_pass(t5, o5)


def _sc_agg(tables, srcv, dstv):
    fn = pl.kernel(
        _sc_agg_body,
        out_type=[jax.ShapeDtypeStruct((N_PAD, C), jnp.float32)] * NCHUNK,
        mesh=plsc.VectorSubcoreMesh(core_axis_name="c", subcore_axis_name="s"),
        scratch_types=[
            pltpu.VMEM_SHARED((N_PAD, C), jnp.float32),
            pltpu.VMEM((SUPER, BLK), jnp.int32),
            pltpu.VMEM((SUPER, BLK), jnp.int32),
            pltpu.VMEM((BLK, C), jnp.float32),
            pltpu.VMEM((BLK, C), jnp.float32),
            pltpu.VMEM((ZROWS, C), jnp.float32),
            pltpu.SemaphoreType.DMA,
            pltpu.SemaphoreType.DMA,
        ],
        compiler_params=pltpu.CompilerParams(use_tc_tiling_on_sc=False),
    )
    return fn(*tables, srcv, dstv)


RB = 1000
NBLK_B = N // RB
RD = 400
NBLK_D = N // RD


def _tc_b_body(x_ref, a0, a1, a2, a3, a4, a5, w1_ref, b1_ref,
               o0, o1, o2, o3, o4, o5):
    agg = jnp.concatenate([a0[...], a1[...], a2[...], a3[...], a4[...],
                           a5[...]], axis=1)[:, :D]
    xa = x_ref[...] + agg
    h = jax.nn.relu(jnp.dot(xa, w1_ref[...],
                            preferred_element_type=jnp.float32) + b1_ref[...])
    hp = jnp.concatenate([h, jnp.zeros((RB, DP - D), jnp.float32)], axis=1)
    outs = (o0, o1, o2, o3, o4, o5)
    for i in range(NCHUNK):
        outs[i][...] = hp[:, C * i:C * (i + 1)]


def _tc_b(x, aggs, W1, b1):
    out = pl.pallas_call(
        _tc_b_body,
        grid=(NBLK_B,),
        in_specs=[pl.BlockSpec((RB, D), lambda i: (i, 0))]
        + [pl.BlockSpec((RB, C), lambda i: (i, 0))] * NCHUNK
        + [pl.BlockSpec((D, D), lambda i: (0, 0)),
           pl.BlockSpec((1, D), lambda i: (0, 0))],
        out_specs=[pl.BlockSpec((RB, C), lambda i: (i, 0))] * NCHUNK,
        out_shape=[jax.ShapeDtypeStruct((N, C), jnp.float32)] * NCHUNK,
    )(x, *aggs, W1, b1.reshape(1, D))
    return out


def _tc_d_body(h0, h1, h2c, h3, h4, h5, a0, a1, a2, a3, a4, a5,
               batch_ref, w2_ref, b2_ref, wg1_ref, bg1_ref, wg2_ref, bg2_ref,
               h2_ref, xg_ref, gmax_acc, gsum_acc):
    i = pl.program_id(0)

    hcat = jnp.concatenate([h0[...], h1[...], h2c[...], h3[...], h4[...],
                            h5[...]], axis=1)[:, :D]
    acat = jnp.concatenate([a0[...], a1[...], a2[...], a3[...], a4[...],
                            a5[...]], axis=1)[:, :D]
    h2blk = jax.nn.relu(jnp.dot(hcat + acat, w2_ref[...],
                                preferred_element_type=jnp.float32)
                        + b2_ref[...])
    h2_ref[...] = h2blk

    @pl.when(i == 0)
    def _():
        gmax_acc[...] = jnp.full((G, 1, D2), -jnp.inf, jnp.float32)
        gsum_acc[...] = jnp.zeros((G, 1, D2 + 1), jnp.float32)

    b = batch_ref[...]
    lo = jnp.min(b)
    hi = jnp.max(b)
    hs = jnp.concatenate([h2blk, jnp.ones((RD, 1), jnp.float32)], axis=1)

    def seg(g, _):
        m = b == g
        mx = jnp.max(jnp.where(m, h2blk, -jnp.inf), axis=0)
        sm = jnp.sum(jnp.where(m, hs, 0.0), axis=0)
        gmax_acc[pl.ds(g, 1)] = jnp.maximum(gmax_acc[pl.ds(g, 1)],
                                            mx[None, None, :])
        gsum_acc[pl.ds(g, 1)] = gsum_acc[pl.ds(g, 1)] + sm[None, None, :]
        return 0

    lax.fori_loop(lo, hi + 1, seg, 0)

    @pl.when(i == NBLK_D - 1)
    def _():
        gmax = gmax_acc[:, 0, :]
        gsum = gsum_acc[:, 0, :D2]
        cnt = gsum_acc[:, 0, D2:]
        gmean = gsum / jnp.maximum(cnt, 1.0)
        wg1 = wg1_ref[...]
        z = (jnp.dot(gmax, wg1[:D2], preferred_element_type=jnp.float32)
             + jnp.dot(gmean, wg1[D2:], preferred_element_type=jnp.float32)
             + bg1_ref[...])
        z = jax.nn.relu(z)
        xg_ref[...] = (jnp.dot(z, wg2_ref[...],
                               preferred_element_type=jnp.float32)
                       + bg2_ref[...])


def _tc_d(hchunks, aggs, batch2d, W2, b2, Wg1, bg1, Wg2, bg2):
    return pl.pallas_call(
        _tc_d_body,
        grid=(NBLK_D,),
        in_specs=[pl.BlockSpec((RD, C), lambda i: (i, 0))] * NCHUNK
        + [pl.BlockSpec((RD, C), lambda i: (i, 0))] * NCHUNK
        + [pl.BlockSpec((RD, 1), lambda i: (i, 0)),
           pl.BlockSpec((D, D2), lambda i: (0, 0)),
           pl.BlockSpec((1, D2), lambda i: (0, 0)),
           pl.BlockSpec((2 * D2, 1024), lambda i: (0, 0)),
           pl.BlockSpec((1, 1024), lambda i: (0, 0)),
           pl.BlockSpec((1024, 384), lambda i: (0, 0)),
           pl.BlockSpec((1, 384), lambda i: (0, 0))],
        out_specs=[pl.BlockSpec((RD, D2), lambda i: (i, 0)),
                   pl.BlockSpec((G, 384), lambda i: (0, 0))],
        out_shape=[jax.ShapeDtypeStruct((N, D2), jnp.float32),
                   jax.ShapeDtypeStruct((G, 384), jnp.float32)],
        scratch_shapes=[pltpu.VMEM((G, 1, D2), jnp.float32),
                        pltpu.VMEM((G, 1, D2 + 1), jnp.float32)],
    )(*hchunks, *aggs, batch2d, W2, b2.reshape(1, D2), Wg1,
      bg1.reshape(1, 1024), Wg2, bg2.reshape(1, 384))


def kernel(x, edge_index, batch, W1, b1, W2, b2, Wg1, bg1, Wg2, bg2):
    src = edge_index[0]
    dst = edge_index[1]
    pad = E_PAD - E
    srcv = jnp.concatenate(
        [src, jnp.arange(pad, dtype=jnp.int32) % 4096]
    ).reshape(E_PAD // BLK, BLK)
    dstv = jnp.concatenate(
        [dst, N + (jnp.arange(pad, dtype=jnp.int32) % 48)]
    ).reshape(E_PAD // BLK, BLK)

    xp = jnp.pad(x, ((0, 0), (0, DP - D)))
    xcs = [xp[:, C * i:C * (i + 1)] for i in range(NCHUNK)]

    agg1 = _sc_agg(xcs, srcv, dstv)
    h1c = _tc_b(x, agg1, W1, b1)
    agg2 = _sc_agg(h1c, srcv, dstv)
    h2, xg = _tc_d(h1c, agg2, batch.reshape(N, 1), W2, b2, Wg1, bg1, Wg2, bg2)
    return (xg, h2, 0)


# speedup vs baseline: 5.2872x; 1.1814x over previous
import jax
import jax.numpy as jnp
from jax import lax
from jax.experimental import pallas as pl
from jax.experimental.pallas import tpu as pltpu
from jax.experimental.pallas import tpu_sc as plsc

N = 50000
E = 800000
G = 128
D = 84
D2 = 840
DP = 96
C = 32
NCHUNK = 3
N_PAD = 50048
NW = N_PAD // 4
BLK = 128
SUPER = 16
E_PAD = 851968
ROWS_PER_CORE = 3328
ROWS_PER_TILE = 208
NSUPER = 13
ZROWS = N_PAD // 16
ZH0 = 1568
ZH1 = ZROWS - ZH0


def _sc_agg_body(t0, t1, t2, srcv, dstv, zeros_h,
                 o00, o01, o02, o10, o11, o12,
                 acc, src_buf, dst_buf, rb0, rb1, rb2, rb3,
                 sg0, sg1, sg2, sg3, ss0, ss1, ss2, ss3):
    c = lax.axis_index("c")
    s = lax.axis_index("s")
    rbufs = (rb0, rb1, rb2, rb3)
    sgs = (sg0, sg1, sg2, sg3)
    sss = (ss0, ss1, ss2, ss3)

    def chunk_pass(table_ref, out_ref, core):
        pltpu.sync_copy(zeros_h, acc.at[pl.ds(s * ZROWS, ZH0)])
        pltpu.sync_copy(zeros_h.at[pl.ds(0, ZH1)],
                        acc.at[pl.ds(s * ZROWS + ZH0, ZH1)])
        plsc.subcore_barrier()

        def super_body(sb, _):
            row0 = core * ROWS_PER_CORE + s * ROWS_PER_TILE + sb * SUPER
            pltpu.sync_copy(srcv.at[pl.ds(row0, SUPER)], src_buf)
            pltpu.sync_copy(dstv.at[pl.ds(row0, SUPER)], dst_buf)
            gd = {}
            sd = {}
            for j in range(2):
                gd[j] = pltpu.async_copy(table_ref.at[src_buf.at[j]],
                                         rbufs[j], sgs[j])
            for j in range(SUPER):
                gd[j].wait()
                sd[j] = pltpu.async_copy(rbufs[j % 4],
                                         acc.at[dst_buf.at[j]],
                                         sss[j % 4], add=True)
                if j + 2 < SUPER:
                    if j >= 2:
                        sd[j - 2].wait()
                    gd[j + 2] = pltpu.async_copy(
                        table_ref.at[src_buf.at[j + 2]],
                        rbufs[(j + 2) % 4], sgs[(j + 2) % 4])
            for j in range(SUPER - 4, SUPER):
                sd[j].wait()
            return 0

        lax.fori_loop(0, NSUPER, super_body, 0)
        plsc.subcore_barrier()
        pltpu.sync_copy(acc.at[pl.ds(s * ZROWS, ZROWS)],
                        out_ref.at[pl.ds(s * ZROWS, ZROWS)])
        plsc.subcore_barrier()

    @pl.when(c == 0)
    def _():
        chunk_pass(t0, o00, 0)
        chunk_pass(t1, o01, 0)
        chunk_pass(t2, o02, 0)

    @pl.when(c == 1)
    def _():
        chunk_pass(t0, o10, 1)
        chunk_pass(t1, o11, 1)
        chunk_pass(t2, o12, 1)


def _sc_agg(tables_wide, srcv, dstv, zeros_h):
    fn = pl.kernel(
        _sc_agg_body,
        out_type=[jax.ShapeDtypeStruct((N_PAD, C), jnp.float32)] * 6,
        mesh=plsc.VectorSubcoreMesh(core_axis_name="c", subcore_axis_name="s"),
        scratch_types=[
            pltpu.VMEM_SHARED((N_PAD, C), jnp.float32),
            pltpu.VMEM((SUPER, BLK), jnp.int32),
            pltpu.VMEM((SUPER, BLK), jnp.int32),
            pltpu.VMEM((BLK, C), jnp.float32),
            pltpu.VMEM((BLK, C), jnp.float32),
            pltpu.VMEM((BLK, C), jnp.float32),
            pltpu.VMEM((BLK, C), jnp.float32),
        ] + [pltpu.SemaphoreType.DMA] * 8,
        compiler_params=pltpu.CompilerParams(use_tc_tiling_on_sc=False),
    )
    narrow = [t.reshape(N_PAD, C) for t in tables_wide]
    outs = fn(*narrow, srcv, dstv, zeros_h)
    return [o.reshape(NW, 128) for o in outs]


RP = 1088
NBLK_P = 46
RB = 544
NBLK_B = 92
RD = 512
NBLK_D = 98


def _unpack(w, rows):
    parts = [w[:, C * a:C * (a + 1)][:, None, :] for a in range(4)]
    return jnp.concatenate(parts, axis=1).reshape(rows, C)


def _pack(a, rows):
    a3 = a.reshape(rows // 4, 4, C)
    return jnp.concatenate([a3[:, i, :] for i in range(4)], axis=1)


def _tc_p_body(x_ref, o0, o1, o2):
    xb = x_ref[...]
    xp = jnp.concatenate([xb, jnp.zeros((RP, DP - D), jnp.float32)], axis=1)
    outs = (o0, o1, o2)
    for k in range(NCHUNK):
        outs[k][...] = _pack(xp[:, C * k:C * (k + 1)], RP)


def _tc_p(x):
    return pl.pallas_call(
        _tc_p_body,
        grid=(NBLK_P,),
        in_specs=[pl.BlockSpec((RP, D), lambda i: (i, 0))],
        out_specs=[pl.BlockSpec((RP // 4, 128), lambda i: (i, 0))] * NCHUNK,
        out_shape=[jax.ShapeDtypeStruct((NW, 128), jnp.float32)] * NCHUNK,
    )(x)


def _tc_b_body(x0, x1, x2, a00, a01, a02, a10, a11, a12, w1_ref, b1_ref,
               o0, o1, o2):
    x96 = jnp.concatenate(
        [_unpack(r[...], RB) for r in (x0, x1, x2)], axis=1)
    a96 = jnp.concatenate(
        [_unpack(a0[...], RB) + _unpack(a1[...], RB)
         for a0, a1 in ((a00, a10), (a01, a11), (a02, a12))], axis=1)
    xa = (x96 + a96)[:, :D]
    h = jax.nn.relu(jnp.dot(xa, w1_ref[...],
                            preferred_element_type=jnp.float32) + b1_ref[...])
    hp = jnp.concatenate([h, jnp.zeros((RB, DP - D), jnp.float32)], axis=1)
    outs = (o0, o1, o2)
    for k in range(NCHUNK):
        outs[k][...] = _pack(hp[:, C * k:C * (k + 1)], RB)


def _tc_b(xw, aggs, W1, b1):
    return pl.pallas_call(
        _tc_b_body,
        grid=(NBLK_B,),
        in_specs=[pl.BlockSpec((RB // 4, 128), lambda i: (i, 0))] * 9
        + [pl.BlockSpec((D, D), lambda i: (0, 0)),
           pl.BlockSpec((1, D), lambda i: (0, 0))],
        out_specs=[pl.BlockSpec((RB // 4, 128), lambda i: (i, 0))] * NCHUNK,
        out_shape=[jax.ShapeDtypeStruct((NW, 128), jnp.float32)] * NCHUNK,
    )(*xw, *aggs, W1, b1.reshape(1, D))


def _tc_d_body(h0, h1, h2c, a00, a01, a02, a10, a11, a12,
               batch_ref, w2_ref, b2_ref, wg1_ref, bg1_ref, wg2_ref, bg2_ref,
               h2t_ref, xg_ref, gmax_acc, gsum_acc):
    i = pl.program_id(0)

    h96 = jnp.concatenate(
        [_unpack(r[...], RD) for r in (h0, h1, h2c)], axis=1)
    a96 = jnp.concatenate(
        [_unpack(a0[...], RD) + _unpack(a1[...], RD)
         for a0, a1 in ((a00, a10), (a01, a11), (a02, a12))], axis=1)
    xa = (h96 + a96)[:, :D]
    h2t = lax.dot_general(w2_ref[...], xa, (((0,), (1,)), ((), ())),
                          preferred_element_type=jnp.float32)
    h2t = jax.nn.relu(h2t + b2_ref[...])
    h2t_ref[...] = h2t

    @pl.when(i == 0)
    def _():
        gmax_acc[...] = jnp.full((G, 1, D2), -jnp.inf, jnp.float32)
        gsum_acc[...] = jnp.zeros((G, 1, D2 + 1), jnp.float32)

    bT = batch_ref[...]
    col = i * RD + lax.broadcasted_iota(jnp.int32, (1, RD), 1)
    valid = col < N
    bT = jnp.clip(bT, 0, G - 1)
    lo = jnp.min(bT)
    hi = jnp.max(bT)
    hst = jnp.concatenate([h2t, jnp.ones((1, RD), jnp.float32)], axis=0)

    def seg(g, _):
        m = (bT == g) & valid
        mx = jnp.max(jnp.where(m, h2t, -jnp.inf), axis=1)
        sm = jnp.sum(jnp.where(m, hst, 0.0), axis=1)
        gmax_acc[pl.ds(g, 1)] = jnp.maximum(gmax_acc[pl.ds(g, 1)],
                                            mx[None, None, :])
        gsum_acc[pl.ds(g, 1)] = gsum_acc[pl.ds(g, 1)] + sm[None, None, :]
        return 0

    lax.fori_loop(lo, hi + 1, seg, 0)

    @pl.when(i == NBLK_D - 1)
    def _():
        gmax = gmax_acc[:, 0, :]
        gsum = gsum_acc[:, 0, :D2]
        cnt = gsum_acc[:, 0, D2:]
        gmean = gsum / jnp.maximum(cnt, 1.0)
        wg1 = wg1_ref[...]
        z = (jnp.dot(gmax, wg1[:D2], preferred_element_type=jnp.float32)
             + jnp.dot(gmean, wg1[D2:], preferred_element_type=jnp.float32)
             + bg1_ref[...])
        z = jax.nn.relu(z)
        xg_ref[...] = (jnp.dot(z, wg2_ref[...],
                               preferred_element_type=jnp.float32)
                       + bg2_ref[...])


def _tc_d(hw, aggs, batch_row, W2, b2, Wg1, bg1, Wg2, bg2):
    return pl.pallas_call(
        _tc_d_body,
        grid=(NBLK_D,),
        in_specs=[pl.BlockSpec((RD // 4, 128), lambda i: (i, 0))] * 9
        + [pl.BlockSpec((1, RD), lambda i: (0, i)),
           pl.BlockSpec((D, D2), lambda i: (0, 0)),
           pl.BlockSpec((D2, 1), lambda i: (0, 0)),
           pl.BlockSpec((2 * D2, 1024), lambda i: (0, 0)),
           pl.BlockSpec((1, 1024), lambda i: (0, 0)),
           pl.BlockSpec((1024, 384), lambda i: (0, 0)),
           pl.BlockSpec((1, 384), lambda i: (0, 0))],
        out_specs=[pl.BlockSpec((D2, RD), lambda i: (0, i)),
                   pl.BlockSpec((G, 384), lambda i: (0, 0))],
        out_shape=[jax.ShapeDtypeStruct((D2, N), jnp.float32),
                   jax.ShapeDtypeStruct((G, 384), jnp.float32)],
        scratch_shapes=[pltpu.VMEM((G, 1, D2), jnp.float32),
                        pltpu.VMEM((G, 1, D2 + 1), jnp.float32)],
    )(*hw, *aggs, batch_row, W2, b2.reshape(D2, 1), Wg1,
      bg1.reshape(1, 1024), Wg2, bg2.reshape(1, 384))


def kernel(x, edge_index, batch, W1, b1, W2, b2, Wg1, bg1, Wg2, bg2):
    src = edge_index[0]
    dst = edge_index[1]
    pad = E_PAD - E
    srcv = jnp.concatenate(
        [src, jnp.arange(pad, dtype=jnp.int32) % 4096]
    ).reshape(E_PAD // BLK, BLK)
    dstv = jnp.concatenate(
        [dst, N + (jnp.arange(pad, dtype=jnp.int32) % 48)]
    ).reshape(E_PAD // BLK, BLK)
    zeros_h = jnp.zeros((ZH0, C), jnp.float32)

    xw = _tc_p(x)
    agg1 = _sc_agg(xw, srcv, dstv, zeros_h)
    h1w = _tc_b(xw, agg1, W1, b1)
    agg2 = _sc_agg(h1w, srcv, dstv, zeros_h)
    h2t, xg = _tc_d(h1w, agg2, batch.reshape(1, N), W2, b2, Wg1, bg1,
                    Wg2, bg2)
    return (xg, h2t.T, 0)


# speedup vs baseline: 7.3626x; 1.3925x over previous
import jax
import jax.numpy as jnp
from jax import lax
from jax.experimental import pallas as pl
from jax.experimental.pallas import tpu as pltpu
from jax.experimental.pallas import tpu_sc as plsc

N = 50000
E = 800000
G = 128
D = 84
D2 = 840
DP = 96
C = 32
NCHUNK = 3
N_PAD = 50048
NW = N_PAD // 4
BLK = 128
SUPER = 16
E_PAD = 851968
ROWS_PER_CORE = 3328
ROWS_PER_TILE = 208
NSUPER = 13
ZROWS = N_PAD // 16
ZH0 = 1568
ZH1 = ZROWS - ZH0


def _sc_agg_body(t0, t1, t2, srcv, dstv, zeros_h,
                 o00, o01, o02, o10, o11, o12,
                 acc, src_buf, dst_buf, rb0, rb1, rb2, rb3,
                 sg0, sg1, sg2, sg3, ss0, ss1, ss2, ss3):
    c = lax.axis_index("c")
    s = lax.axis_index("s")
    rbufs = (rb0, rb1, rb2, rb3)
    sgs = (sg0, sg1, sg2, sg3)
    sss = (ss0, ss1, ss2, ss3)

    def chunk_pass(table_ref, out_ref, core):
        pltpu.sync_copy(zeros_h, acc.at[pl.ds(s * ZROWS, ZH0)])
        pltpu.sync_copy(zeros_h.at[pl.ds(0, ZH1)],
                        acc.at[pl.ds(s * ZROWS + ZH0, ZH1)])
        plsc.subcore_barrier()

        def super_body(sb, _):
            row0 = core * ROWS_PER_CORE + s * ROWS_PER_TILE + sb * SUPER
            pltpu.sync_copy(srcv.at[pl.ds(row0, SUPER)], src_buf)
            pltpu.sync_copy(dstv.at[pl.ds(row0, SUPER)], dst_buf)
            gd = {}
            sd = {}
            for j in range(2):
                gd[j] = pltpu.async_copy(table_ref.at[src_buf.at[j]],
                                         rbufs[j], sgs[j])
            for j in range(SUPER):
                gd[j].wait()
                sd[j] = pltpu.async_copy(rbufs[j % 4],
                                         acc.at[dst_buf.at[j]],
                                         sss[j % 4], add=True)
                if j + 2 < SUPER:
                    if j >= 2:
                        sd[j - 2].wait()
                    gd[j + 2] = pltpu.async_copy(
                        table_ref.at[src_buf.at[j + 2]],
                        rbufs[(j + 2) % 4], sgs[(j + 2) % 4])
            for j in range(SUPER - 4, SUPER):
                sd[j].wait()
            return 0

        lax.fori_loop(0, NSUPER, super_body, 0)
        plsc.subcore_barrier()
        pltpu.sync_copy(acc.at[pl.ds(s * ZROWS, ZROWS)],
                        out_ref.at[pl.ds(s * ZROWS, ZROWS)])
        plsc.subcore_barrier()

    @pl.when(c == 0)
    def _():
        chunk_pass(t0, o00, 0)
        chunk_pass(t1, o01, 0)
        chunk_pass(t2, o02, 0)

    @pl.when(c == 1)
    def _():
        chunk_pass(t0, o10, 1)
        chunk_pass(t1, o11, 1)
        chunk_pass(t2, o12, 1)


def _sc_agg(tables_wide, srcv, dstv, zeros_h):
    fn = pl.kernel(
        _sc_agg_body,
        out_type=[jax.ShapeDtypeStruct((N_PAD, C), jnp.float32)] * 6,
        mesh=plsc.VectorSubcoreMesh(core_axis_name="c", subcore_axis_name="s"),
        scratch_types=[
            pltpu.VMEM_SHARED((N_PAD, C), jnp.float32),
            pltpu.VMEM((SUPER, BLK), jnp.int32),
            pltpu.VMEM((SUPER, BLK), jnp.int32),
            pltpu.VMEM((BLK, C), jnp.float32),
            pltpu.VMEM((BLK, C), jnp.float32),
            pltpu.VMEM((BLK, C), jnp.float32),
            pltpu.VMEM((BLK, C), jnp.float32),
        ] + [pltpu.SemaphoreType.DMA] * 8,
        compiler_params=pltpu.CompilerParams(use_tc_tiling_on_sc=False),
    )
    narrow = [t.reshape(N_PAD, C) for t in tables_wide]
    outs = fn(*narrow, srcv, dstv, zeros_h)
    return [o.reshape(NW, 128) for o in outs]


RP = 1088
NBLK_P = 46
RB = 544
NBLK_B = 92
RD = 512
NBLK_D = 98


def _unpack(w, rows):
    parts = [w[:, C * a:C * (a + 1)][:, None, :] for a in range(4)]
    return jnp.concatenate(parts, axis=1).reshape(rows, C)


def _pack(a, rows):
    a3 = a.reshape(rows // 4, 4, C)
    return jnp.concatenate([a3[:, i, :] for i in range(4)], axis=1)


def _tc_p_body(x_ref, o0, o1, o2):
    xb = x_ref[...]
    xp = jnp.concatenate([xb, jnp.zeros((RP, DP - D), jnp.float32)], axis=1)
    outs = (o0, o1, o2)
    for k in range(NCHUNK):
        outs[k][...] = _pack(xp[:, C * k:C * (k + 1)], RP)


def _tc_p(x):
    return pl.pallas_call(
        _tc_p_body,
        grid=(NBLK_P,),
        in_specs=[pl.BlockSpec((RP, D), lambda i: (i, 0))],
        out_specs=[pl.BlockSpec((RP // 4, 128), lambda i: (i, 0))] * NCHUNK,
        out_shape=[jax.ShapeDtypeStruct((NW, 128), jnp.float32)] * NCHUNK,
    )(x)


def _tc_b_body(x_ref, a00, a01, a02, a10, a11, a12, w1_ref, b1_ref,
               o0, o1, o2, oh):
    a96 = jnp.concatenate(
        [_unpack(a0[...] + a1[...], RB)
         for a0, a1 in ((a00, a10), (a01, a11), (a02, a12))], axis=1)
    xa = x_ref[...] + a96[:, :D]
    h = jax.nn.relu(jnp.dot(xa, w1_ref[...],
                            preferred_element_type=jnp.float32) + b1_ref[...])
    oh[...] = h
    hp = jnp.concatenate([h, jnp.zeros((RB, DP - D), jnp.float32)], axis=1)
    outs = (o0, o1, o2)
    for k in range(NCHUNK):
        outs[k][...] = _pack(hp[:, C * k:C * (k + 1)], RB)


def _tc_b(x, aggs, W1, b1):
    return pl.pallas_call(
        _tc_b_body,
        grid=(NBLK_B,),
        in_specs=[pl.BlockSpec((RB, D), lambda i: (i, 0))]
        + [pl.BlockSpec((RB // 4, 128), lambda i: (i, 0))] * 6
        + [pl.BlockSpec((D, D), lambda i: (0, 0)),
           pl.BlockSpec((1, D), lambda i: (0, 0))],
        out_specs=[pl.BlockSpec((RB // 4, 128), lambda i: (i, 0))] * NCHUNK
        + [pl.BlockSpec((RB, D), lambda i: (i, 0))],
        out_shape=[jax.ShapeDtypeStruct((NW, 128), jnp.float32)] * NCHUNK
        + [jax.ShapeDtypeStruct((N, D), jnp.float32)],
    )(x, *aggs, W1, b1.reshape(1, D))


def _tc_d_body(h_ref, a00, a01, a02, a10, a11, a12,
               batch_ref, w2_ref, b2_ref, wg1_ref, bg1_ref, wg2_ref, bg2_ref,
               h2t_ref, xg_ref, gmax_acc, gsum_acc):
    i = pl.program_id(0)

    a96 = jnp.concatenate(
        [_unpack(a0[...] + a1[...], RD)
         for a0, a1 in ((a00, a10), (a01, a11), (a02, a12))], axis=1)
    xa = h_ref[...] + a96[:, :D]
    h2t = lax.dot_general(w2_ref[...], xa, (((0,), (1,)), ((), ())),
                          preferred_element_type=jnp.float32)
    h2t = jax.nn.relu(h2t + b2_ref[...])
    h2t_ref[...] = h2t

    @pl.when(i == 0)
    def _():
        gmax_acc[...] = jnp.full((G, 1, D2), -jnp.inf, jnp.float32)
        gsum_acc[...] = jnp.zeros((G, 1, D2 + 1), jnp.float32)

    bT = batch_ref[...]
    col = i * RD + lax.broadcasted_iota(jnp.int32, (1, RD), 1)
    valid = col < N
    bT = jnp.clip(bT, 0, G - 1)
    lo = jnp.min(bT)
    hi = jnp.max(bT)
    hst = jnp.concatenate([h2t, jnp.ones((1, RD), jnp.float32)], axis=0)

    def seg(g, _):
        m = (bT == g) & valid
        mx = jnp.max(jnp.where(m, h2t, -jnp.inf), axis=1)
        sm = jnp.sum(jnp.where(m, hst, 0.0), axis=1)
        gmax_acc[pl.ds(g, 1)] = jnp.maximum(gmax_acc[pl.ds(g, 1)],
                                            mx[None, None, :])
        gsum_acc[pl.ds(g, 1)] = gsum_acc[pl.ds(g, 1)] + sm[None, None, :]
        return 0

    lax.fori_loop(lo, hi + 1, seg, 0)

    @pl.when(i == NBLK_D - 1)
    def _():
        gmax = gmax_acc[:, 0, :]
        gsum = gsum_acc[:, 0, :D2]
        cnt = gsum_acc[:, 0, D2:]
        gmean = gsum / jnp.maximum(cnt, 1.0)
        wg1 = wg1_ref[...]
        z = (jnp.dot(gmax, wg1[:D2], preferred_element_type=jnp.float32)
             + jnp.dot(gmean, wg1[D2:], preferred_element_type=jnp.float32)
             + bg1_ref[...])
        z = jax.nn.relu(z)
        xg_ref[...] = (jnp.dot(z, wg2_ref[...],
                               preferred_element_type=jnp.float32)
                       + bg2_ref[...])


def _tc_d(h1, aggs, batch_row, W2, b2, Wg1, bg1, Wg2, bg2):
    return pl.pallas_call(
        _tc_d_body,
        grid=(NBLK_D,),
        in_specs=[pl.BlockSpec((RD, D), lambda i: (i, 0))]
        + [pl.BlockSpec((RD // 4, 128), lambda i: (i, 0))] * 6
        + [pl.BlockSpec((1, RD), lambda i: (0, i)),
           pl.BlockSpec((D, D2), lambda i: (0, 0)),
           pl.BlockSpec((D2, 1), lambda i: (0, 0)),
           pl.BlockSpec((2 * D2, 1024), lambda i: (0, 0)),
           pl.BlockSpec((1, 1024), lambda i: (0, 0)),
           pl.BlockSpec((1024, 384), lambda i: (0, 0)),
           pl.BlockSpec((1, 384), lambda i: (0, 0))],
        out_specs=[pl.BlockSpec((D2, RD), lambda i: (0, i)),
                   pl.BlockSpec((G, 384), lambda i: (0, 0))],
        out_shape=[jax.ShapeDtypeStruct((D2, N), jnp.float32),
                   jax.ShapeDtypeStruct((G, 384), jnp.float32)],
        scratch_shapes=[pltpu.VMEM((G, 1, D2), jnp.float32),
                        pltpu.VMEM((G, 1, D2 + 1), jnp.float32)],
    )(h1, *aggs, batch_row, W2, b2.reshape(D2, 1), Wg1,
      bg1.reshape(1, 1024), Wg2, bg2.reshape(1, 384))


def kernel(x, edge_index, batch, W1, b1, W2, b2, Wg1, bg1, Wg2, bg2):
    src = edge_index[0]
    dst = edge_index[1]
    pad = E_PAD - E
    srcv = jnp.concatenate(
        [src, jnp.arange(pad, dtype=jnp.int32) % 4096]
    ).reshape(E_PAD // BLK, BLK)
    dstv = jnp.concatenate(
        [dst, N + (jnp.arange(pad, dtype=jnp.int32) % 48)]
    ).reshape(E_PAD // BLK, BLK)
    zeros_h = jnp.zeros((ZH0, C), jnp.float32)

    xw = _tc_p(x)
    agg1 = _sc_agg(xw, srcv, dstv, zeros_h)
    *h1w, h1 = _tc_b(x, agg1, W1, b1)
    agg2 = _sc_agg(h1w, srcv, dstv, zeros_h)
    h2t, xg = _tc_d(h1, agg2, batch.reshape(1, N), W2, b2, Wg1, bg1,
                    Wg2, bg2)
    return (xg, h2t.T, 0)
